# baseline (speedup 1.0000x reference)
import jax
import jax.numpy as jnp
from jax.experimental import pallas as pl
from jax.experimental.pallas import tpu as pltpu

N_PIX = 262144
K = 1024
BLOCK = 8192


def _vq_body(xt_ref, wt_ref, ckct_ref, o_ref):
    x0 = xt_ref[0:1, :]
    x1 = xt_ref[1:2, :]
    x2 = xt_ref[2:3, :]
    d = ((wt_ref[:, 0:1] * x0 + wt_ref[:, 3:4])
         + (wt_ref[:, 1:2] * x1 + wt_ref[:, 2:3] * x2))
    m = jnp.min(d, axis=0, keepdims=True)
    onehot = (d <= m).astype(jnp.float32)
    g = jnp.dot(ckct_ref[...], onehot,
                preferred_element_type=jnp.float32)
    o_ref[...] = g[0:3, :] / g[3:4, :]


def kernel(x, cluster_centers):
    ccsq = jnp.sum(cluster_centers * cluster_centers, axis=1)
    wt = jnp.concatenate(
        [-2.0 * cluster_centers, ccsq[:, None]], axis=1)
    ckct = jnp.concatenate(
        [cluster_centers.T, jnp.ones((1, K), jnp.float32)], axis=0)
    xt = x.T
    grid = (N_PIX // BLOCK,)
    out_t = pl.pallas_call(
        _vq_body,
        grid=grid,
        in_specs=[
            pl.BlockSpec((3, BLOCK), lambda i: (0, i)),
            pl.BlockSpec((K, 4), lambda i: (0, 0)),
            pl.BlockSpec((4, K), lambda i: (0, 0)),
        ],
        out_specs=pl.BlockSpec((3, BLOCK), lambda i: (0, i)),
        out_shape=jax.ShapeDtypeStruct((3, N_PIX), jnp.float32),
        compiler_params=pltpu.CompilerParams(
            dimension_semantics=("parallel",),
        ),
    )(xt, wt, ckct)
    return out_t.T


# speedup vs baseline: 1.0012x; 1.0012x over previous
import jax
import jax.numpy as jnp
from jax.experimental import pallas as pl
from jax.experimental.pallas import tpu as pltpu

N_PIX = 262144
K = 1024
BLOCK = 8192


def _vq_body(xt_ref, wt_ref, ckct_ref, o_ref):
    x0 = xt_ref[0:1, :]
    x1 = xt_ref[1:2, :]
    x2 = xt_ref[2:3, :]
    d = ((wt_ref[:, 0:1] * x0 + wt_ref[:, 3:4])
         + (wt_ref[:, 1:2] * x1 + wt_ref[:, 2:3] * x2))
    m = jnp.min(d, axis=0, keepdims=True)
    onehot = (d <= m).astype(jnp.bfloat16)
    g = jnp.dot(ckct_ref[...], onehot,
                preferred_element_type=jnp.float32)
    o_ref[...] = g[0:3, :] / g[3:4, :]


def kernel(x, cluster_centers):
    ccsq = jnp.sum(cluster_centers * cluster_centers, axis=1)
    wt = jnp.concatenate(
        [-2.0 * cluster_centers, ccsq[:, None]], axis=1)
    ckct = jnp.concatenate(
        [cluster_centers.T, jnp.ones((1, K), jnp.float32)], axis=0)
    xt = x.T
    grid = (N_PIX // BLOCK,)
    out_t = pl.pallas_call(
        _vq_body,
        grid=grid,
        in_specs=[
            pl.BlockSpec((3, BLOCK), lambda i: (0, i)),
            pl.BlockSpec((K, 4), lambda i: (0, 0)),
            pl.BlockSpec((4, K), lambda i: (0, 0)),
        ],
        out_specs=pl.BlockSpec((3, BLOCK), lambda i: (0, i)),
        out_shape=jax.ShapeDtypeStruct((3, N_PIX), jnp.float32),
        compiler_params=pltpu.CompilerParams(
            dimension_semantics=("parallel",),
        ),
    )(xt, wt, ckct)
    return out_t.T
